# Initial kernel scaffold; baseline (speedup 1.0000x reference)
import jax
import jax.numpy as jnp
from jax.experimental import pallas as pl


def kernel(tile_feat, rr_feat, t2t_src, t2t_dst, r2t_src, r2t_dst, t2r_src, t2r_dst, is_sink, params):
    raise NotImplementedError("write your pallas kernel here")


# speedup vs baseline: 2.1290x; 2.1290x over previous
import functools
import jax
import jax.numpy as jnp
from jax.experimental import pallas as pl

_N_TILE = 50000
_N_RR = 10000
_D = 128
_H = 128
_TAU = 0.1
_K = 5000
_TE = 2000
_RT = 1024
_NS = 10240


def _proj_body(x_ref, m_ref, w_ref, c_ref, o_ref):
    x = x_ref[...]
    num = x[:, :128]
    den = jnp.maximum(x[:, 128:129], 1e-20)
    feat = (num / den) * m_ref[...]
    o_ref[...] = jnp.dot(feat, w_ref[...],
                         preferred_element_type=jnp.float32) + c_ref[...]


def _proj(xfull, m, w, c):
    n = xfull.shape[0]
    grid = n // _TE
    return pl.pallas_call(
        _proj_body,
        grid=(grid,),
        in_specs=[
            pl.BlockSpec((_TE, 144), lambda i: (i, 0)),
            pl.BlockSpec((_TE, 1), lambda i: (i, 0)),
            pl.BlockSpec((128, 128), lambda i: (0, 0)),
            pl.BlockSpec((1, 128), lambda i: (0, 0)),
        ],
        out_specs=pl.BlockSpec((_TE, 128), lambda i: (i, 0)),
        out_shape=jax.ShapeDtypeStruct((n, 128), jnp.float32),
    )(xfull, m, w, c)


def _edge_mlp_body(g_ref, w2_ref, b2_ref, o_ref):
    h = jnp.maximum(g_ref[...], 0.0)
    msg = jnp.dot(h, w2_ref[...],
                  preferred_element_type=jnp.float32) + b2_ref[...]
    nrm = jnp.sqrt(jnp.sum(msg * msg, axis=1, keepdims=True))
    wgt = jnp.exp(nrm * (-1.0 / _TAU))
    out = jnp.concatenate(
        [msg * wgt, wgt, jnp.zeros((msg.shape[0], 15), jnp.float32)], axis=1)
    o_ref[...] = out


def _edge_mlp(g, w2, b2):
    e = g.shape[0]
    grid = e // _TE
    return pl.pallas_call(
        _edge_mlp_body,
        grid=(grid,),
        in_specs=[
            pl.BlockSpec((_TE, 128), lambda i: (i, 0)),
            pl.BlockSpec((128, 128), lambda i: (0, 0)),
            pl.BlockSpec((1, 128), lambda i: (0, 0)),
        ],
        out_specs=pl.BlockSpec((_TE, 144), lambda i: (i, 0)),
        out_shape=jax.ShapeDtypeStruct((e, 144), jnp.float32),
    )(g, w2, b2)


def _rank_body(sa_ref, sb_ref, o_ref):
    i = pl.program_id(0)
    j = pl.program_id(1)

    @pl.when(j == 0)
    def _():
        o_ref[...] = jnp.zeros_like(o_ref)

    a = sa_ref[...]
    b = sb_ref[0]
    ia = i * _RT + jax.lax.broadcasted_iota(jnp.int32, (_RT, 1), 0)
    jb = j * _RT + jax.lax.broadcasted_iota(jnp.int32, (1, _RT), 1)
    gt = b > a
    tie = (b == a) & (jb < ia)
    cnt = jnp.sum((gt | tie).astype(jnp.int32), axis=1, keepdims=True)
    o_ref[...] += cnt


def _rank(scores_pad):
    sa = scores_pad.reshape(_NS, 1)
    sb = scores_pad.reshape(_NS // _RT, 1, _RT)
    grid = _NS // _RT
    return pl.pallas_call(
        _rank_body,
        grid=(grid, grid),
        in_specs=[
            pl.BlockSpec((_RT, 1), lambda i, j: (i, 0)),
            pl.BlockSpec((1, 1, _RT), lambda i, j: (j, 0, 0)),
        ],
        out_specs=pl.BlockSpec((_RT, 1), lambda i, j: (i, 0)),
        out_shape=jax.ShapeDtypeStruct((_NS, 1), jnp.int32),
    )(sa, sb)


def _sum_body(x_ref, m_ref, o_ref):
    i = pl.program_id(0)

    @pl.when(i == 0)
    def _():
        o_ref[...] = jnp.zeros_like(o_ref)

    x = x_ref[...]
    num = x[:, :128]
    den = jnp.maximum(x[:, 128:129], 1e-20)
    feat = (num / den) * m_ref[...]
    o_ref[...] += jnp.sum(feat, axis=0, keepdims=True)


def _masked_sum(xfull, m):
    n = xfull.shape[0]
    grid = n // _TE
    return pl.pallas_call(
        _sum_body,
        grid=(grid,),
        in_specs=[
            pl.BlockSpec((_TE, 144), lambda i: (i, 0)),
            pl.BlockSpec((_TE, 1), lambda i: (i, 0)),
        ],
        out_specs=pl.BlockSpec((1, 128), lambda i: (0, 0)),
        out_shape=jax.ShapeDtypeStruct((1, 128), jnp.float32),
    )(xfull, m)


def _layer(src_full, src_m, dst_full, dst_m, src_idx, dst_idx, n_out,
           w1a, w1b, b1, w2, b2):
    ps = _proj(src_full, src_m, w1a, jnp.zeros((1, 128), jnp.float32))
    pd = _proj(dst_full, dst_m, w1b, b1)
    g = jnp.take(ps, src_idx, axis=0) + jnp.take(pd, dst_idx, axis=0)
    wm = _edge_mlp(g, w2, b2)
    acc = jax.ops.segment_sum(wm, dst_idx, num_segments=n_out)
    return acc


def kernel(tile_feat, rr_feat, t2t_src, t2t_dst, r2t_src, r2t_dst,
           t2r_src, t2r_dst, is_sink, params):
    p = params
    f32 = jnp.float32
    tile0 = jnp.concatenate(
        [tile_feat, jnp.ones((_N_TILE, 1), f32),
         jnp.zeros((_N_TILE, 15), f32)], axis=1)
    rr0 = jnp.concatenate(
        [rr_feat, jnp.ones((_N_RR, 1), f32),
         jnp.zeros((_N_RR, 15), f32)], axis=1)
    ones_t = jnp.ones((_N_TILE, 1), f32)
    ones_r = jnp.ones((_N_RR, 1), f32)

    def wsplit(nm):
        w1 = p[nm + '_w1']
        return (w1[:_D], w1[_D:], p[nm + '_b1'].reshape(1, _H),
                p[nm + '_w2'], p[nm + '_b2'].reshape(1, _H))

    a1, b1_, c1, d1, e1 = wsplit('t2t')
    tile1 = _layer(tile0, ones_t, tile0, ones_t, t2t_src, t2t_dst,
                   _N_TILE, a1, b1_, c1, d1, e1)
    a2, b2_, c2, d2, e2 = wsplit('r2t')
    tile2 = _layer(rr0, ones_r, tile1, ones_t, r2t_src, r2t_dst,
                   _N_TILE, a2, b2_, c2, d2, e2)
    a3, b3_, c3, d3, e3 = wsplit('t2r')
    rr1 = _layer(tile2, ones_t, rr0, ones_r, t2r_src, t2r_dst,
                 _N_RR, a3, b3_, c3, d3, e3)

    sw = jnp.zeros((128, 128), f32).at[:, 0:1].set(p['score_w'])
    sb = jnp.zeros((1, 128), f32).at[0, 0].set(p['score_b'][0])
    scores = _proj(rr1, ones_r, sw, sb)[:, 0]
    scores_pad = jnp.concatenate(
        [scores, jnp.full((_NS - _N_RR,), -jnp.inf, f32)])
    rank = _rank(scores_pad)[:_N_RR, :]
    mask = (rank < _K).astype(f32)

    a4, b4_, c4, d4, e4 = wsplit('pt2t')
    tile3 = _layer(tile2, ones_t, tile2, ones_t, t2t_src, t2t_dst,
                   _N_TILE, a4, b4_, c4, d4, e4)
    a5, b5_, c5, d5, e5 = wsplit('pr2t')
    tile4 = _layer(rr1, mask, tile3, ones_t, r2t_src, r2t_dst,
                   _N_TILE, a5, b5_, c5, d5, e5)
    a6, b6_, c6, d6, e6 = wsplit('pt2r')
    rr2 = _layer(tile4, ones_t, rr1, mask, t2r_src, t2r_dst,
                 _N_RR, a6, b6_, c6, d6, e6)

    sink_m = is_sink.astype(f32).reshape(_N_RR, 1)
    sink_feat = _masked_sum(rr2, sink_m)
    tile_read = _masked_sum(tile4, ones_t)
    return jnp.concatenate([sink_feat, tile_read], axis=-1)


# speedup vs baseline: 3.3249x; 1.5617x over previous
import functools
import jax
import jax.numpy as jnp
from jax import lax
from jax.experimental import pallas as pl
from jax.experimental.pallas import tpu as pltpu
from jax.experimental.pallas import tpu_sc as plsc

_N_TILE = 50000
_N_RR = 10000
_D = 128
_H = 128
_TAU = 0.1
_K = 5000
_TE = 2000
_RT = 1024
_NS = 10240


def _proj_body(x_ref, m_ref, w_ref, c_ref, o_ref):
    x = x_ref[...]
    num = x[:, :128]
    den = jnp.maximum(x[:, 128:129], 1e-20)
    feat = (num / den) * m_ref[...]
    o_ref[...] = jnp.dot(feat, w_ref[...],
                         preferred_element_type=jnp.float32) + c_ref[...]


def _proj(xfull, m, w, c):
    n = xfull.shape[0]
    grid = n // _TE
    return pl.pallas_call(
        _proj_body,
        grid=(grid,),
        in_specs=[
            pl.BlockSpec((_TE, 144), lambda i: (i, 0)),
            pl.BlockSpec((_TE, 1), lambda i: (i, 0)),
            pl.BlockSpec((128, 128), lambda i: (0, 0)),
            pl.BlockSpec((1, 128), lambda i: (0, 0)),
        ],
        out_specs=pl.BlockSpec((_TE, 128), lambda i: (i, 0)),
        out_shape=jax.ShapeDtypeStruct((n, 128), jnp.float32),
    )(xfull, m, w, c)


def _edge_mlp_body(g_ref, w2_ref, b2_ref, o_ref):
    h = jnp.maximum(g_ref[...], 0.0)
    msg = jnp.dot(h, w2_ref[...],
                  preferred_element_type=jnp.float32) + b2_ref[...]
    nrm = jnp.sqrt(jnp.sum(msg * msg, axis=1, keepdims=True))
    wgt = jnp.exp(nrm * (-1.0 / _TAU))
    out = jnp.concatenate(
        [msg * wgt, wgt, jnp.zeros((msg.shape[0], 15), jnp.float32)], axis=1)
    o_ref[...] = out


def _edge_mlp(g, w2, b2):
    e = g.shape[0]
    grid = e // _TE
    return pl.pallas_call(
        _edge_mlp_body,
        grid=(grid,),
        in_specs=[
            pl.BlockSpec((_TE, 128), lambda i: (i, 0)),
            pl.BlockSpec((128, 128), lambda i: (0, 0)),
            pl.BlockSpec((1, 128), lambda i: (0, 0)),
        ],
        out_specs=pl.BlockSpec((_TE, 144), lambda i: (i, 0)),
        out_shape=jax.ShapeDtypeStruct((e, 144), jnp.float32),
    )(g, w2, b2)


def _rank_body(sa_ref, sb_ref, o_ref):
    i = pl.program_id(0)
    j = pl.program_id(1)

    @pl.when(j == 0)
    def _():
        o_ref[...] = jnp.zeros_like(o_ref)

    a = sa_ref[...]
    b = sb_ref[0]
    ia = i * _RT + jax.lax.broadcasted_iota(jnp.int32, (_RT, 1), 0)
    jb = j * _RT + jax.lax.broadcasted_iota(jnp.int32, (1, _RT), 1)
    gt = b > a
    tie = (b == a) & (jb < ia)
    cnt = jnp.sum((gt | tie).astype(jnp.int32), axis=1, keepdims=True)
    o_ref[...] += cnt


def _rank(scores_pad):
    sa = scores_pad.reshape(_NS, 1)
    sb = scores_pad.reshape(_NS // _RT, 1, _RT)
    grid = _NS // _RT
    return pl.pallas_call(
        _rank_body,
        grid=(grid, grid),
        in_specs=[
            pl.BlockSpec((_RT, 1), lambda i, j: (i, 0)),
            pl.BlockSpec((1, 1, _RT), lambda i, j: (j, 0, 0)),
        ],
        out_specs=pl.BlockSpec((_RT, 1), lambda i, j: (i, 0)),
        out_shape=jax.ShapeDtypeStruct((_NS, 1), jnp.int32),
    )(sa, sb)


def _sum_body(x_ref, m_ref, o_ref):
    i = pl.program_id(0)

    @pl.when(i == 0)
    def _():
        o_ref[...] = jnp.zeros_like(o_ref)

    x = x_ref[...]
    num = x[:, :128]
    den = jnp.maximum(x[:, 128:129], 1e-20)
    feat = (num / den) * m_ref[...]
    o_ref[...] += jnp.sum(feat, axis=0, keepdims=True)


def _masked_sum(xfull, m):
    n = xfull.shape[0]
    grid = n // _TE
    return pl.pallas_call(
        _sum_body,
        grid=(grid,),
        in_specs=[
            pl.BlockSpec((_TE, 144), lambda i: (i, 0)),
            pl.BlockSpec((_TE, 1), lambda i: (i, 0)),
        ],
        out_specs=pl.BlockSpec((1, 128), lambda i: (0, 0)),
        out_shape=jax.ShapeDtypeStruct((1, 128), jnp.float32),
    )(xfull, m)


_GCH = 400
_GSUB = 80
_NC = 2
_NSUB = 16
_NW = _NC * _NSUB


def _sc_gather(ps, pd, src_idx, dst_idx):
    e = src_idx.shape[0]
    nchunks = e // _GCH
    per_w = -(-nchunks // _NW)
    mesh = plsc.VectorSubcoreMesh(core_axis_name="c", subcore_axis_name="s")

    @functools.partial(
        pl.kernel, mesh=mesh,
        out_type=jax.ShapeDtypeStruct((e, 128), jnp.float32),
        scratch_types=[
            pltpu.VMEM((_GCH,), jnp.int32),
            pltpu.VMEM((_GCH,), jnp.int32),
            pltpu.VMEM((_GCH, 128), jnp.float32),
            pltpu.VMEM((_GCH, 128), jnp.float32),
            pltpu.SemaphoreType.DMA,
        ],
    )
    def k(ps_hbm, pd_hbm, si_hbm, di_hbm, out_hbm, ia, ib, ra, rb, sem):
        wid = lax.axis_index("s") * _NC + lax.axis_index("c")

        def body(t, carry):
            c = t * _NW + wid

            @pl.when(c < nchunks)
            def _():
                base = c * _GCH
                pltpu.sync_copy(si_hbm.at[pl.ds(base, _GCH)], ia)
                pltpu.sync_copy(di_hbm.at[pl.ds(base, _GCH)], ib)
                cps = []
                for q in range(_GCH // _GSUB):
                    sl = pl.ds(q * _GSUB, _GSUB)
                    cps.append(pltpu.async_copy(
                        ps_hbm.at[ia.at[sl]], ra.at[sl], sem))
                    cps.append(pltpu.async_copy(
                        pd_hbm.at[ib.at[sl]], rb.at[sl], sem))
                for cp in cps:
                    cp.wait()

                def add_row(r, cr):
                    for j in range(8):
                        s2 = pl.ds(j * 16, 16)
                        ra[r, s2] = ra[r, s2] + rb[r, s2]
                    return cr

                lax.fori_loop(0, _GCH, add_row, 0)
                pltpu.sync_copy(ra, out_hbm.at[pl.ds(base, _GCH)])

            return carry

        lax.fori_loop(0, per_w, body, 0)

    return k(ps, pd, src_idx, dst_idx)


def _layer(src_full, src_m, dst_full, dst_m, src_idx, dst_idx, n_out,
           w1a, w1b, b1, w2, b2):
    ps = _proj(src_full, src_m, w1a, jnp.zeros((1, 128), jnp.float32))
    pd = _proj(dst_full, dst_m, w1b, b1)
    g = _sc_gather(ps, pd, src_idx, dst_idx)
    wm = _edge_mlp(g, w2, b2)
    acc = jax.ops.segment_sum(wm, dst_idx, num_segments=n_out)
    return acc


def kernel(tile_feat, rr_feat, t2t_src, t2t_dst, r2t_src, r2t_dst,
           t2r_src, t2r_dst, is_sink, params):
    p = params
    f32 = jnp.float32
    i32 = jnp.int32
    t2t_src, t2t_dst = t2t_src.astype(i32), t2t_dst.astype(i32)
    r2t_src, r2t_dst = r2t_src.astype(i32), r2t_dst.astype(i32)
    t2r_src, t2r_dst = t2r_src.astype(i32), t2r_dst.astype(i32)
    tile0 = jnp.concatenate(
        [tile_feat, jnp.ones((_N_TILE, 1), f32),
         jnp.zeros((_N_TILE, 15), f32)], axis=1)
    rr0 = jnp.concatenate(
        [rr_feat, jnp.ones((_N_RR, 1), f32),
         jnp.zeros((_N_RR, 15), f32)], axis=1)
    ones_t = jnp.ones((_N_TILE, 1), f32)
    ones_r = jnp.ones((_N_RR, 1), f32)

    def wsplit(nm):
        w1 = p[nm + '_w1']
        return (w1[:_D], w1[_D:], p[nm + '_b1'].reshape(1, _H),
                p[nm + '_w2'], p[nm + '_b2'].reshape(1, _H))

    a1, b1_, c1, d1, e1 = wsplit('t2t')
    tile1 = _layer(tile0, ones_t, tile0, ones_t, t2t_src, t2t_dst,
                   _N_TILE, a1, b1_, c1, d1, e1)
    a2, b2_, c2, d2, e2 = wsplit('r2t')
    tile2 = _layer(rr0, ones_r, tile1, ones_t, r2t_src, r2t_dst,
                   _N_TILE, a2, b2_, c2, d2, e2)
    a3, b3_, c3, d3, e3 = wsplit('t2r')
    rr1 = _layer(tile2, ones_t, rr0, ones_r, t2r_src, t2r_dst,
                 _N_RR, a3, b3_, c3, d3, e3)

    sw = jnp.zeros((128, 128), f32).at[:, 0:1].set(p['score_w'])
    sb = jnp.zeros((1, 128), f32).at[0, 0].set(p['score_b'][0])
    scores = _proj(rr1, ones_r, sw, sb)[:, 0]
    scores_pad = jnp.concatenate(
        [scores, jnp.full((_NS - _N_RR,), -jnp.inf, f32)])
    rank = _rank(scores_pad)[:_N_RR, :]
    mask = (rank < _K).astype(f32)

    a4, b4_, c4, d4, e4 = wsplit('pt2t')
    tile3 = _layer(tile2, ones_t, tile2, ones_t, t2t_src, t2t_dst,
                   _N_TILE, a4, b4_, c4, d4, e4)
    a5, b5_, c5, d5, e5 = wsplit('pr2t')
    tile4 = _layer(rr1, mask, tile3, ones_t, r2t_src, r2t_dst,
                   _N_TILE, a5, b5_, c5, d5, e5)
    a6, b6_, c6, d6, e6 = wsplit('pt2r')
    rr2 = _layer(tile4, ones_t, rr1, mask, t2r_src, t2r_dst,
                 _N_RR, a6, b6_, c6, d6, e6)

    sink_m = is_sink.astype(f32).reshape(_N_RR, 1)
    sink_feat = _masked_sum(rr2, sink_m)
    tile_read = _masked_sum(tile4, ones_t)
    return jnp.concatenate([sink_feat, tile_read], axis=-1)
